# baseline (speedup 1.0000x reference)
import functools

import jax
import jax.numpy as jnp
from jax import lax
from jax.experimental import pallas as pl
from jax.experimental.pallas import tpu as pltpu
from jax.experimental.pallas import tpu_sc as plsc

_NTOKEN = 100000
_NINP = 300
_BATCH = 16384
_SEQ = 50

_NC = 2
_NS = 16
_NW = _NC * _NS

_DP = 304
_PW = 2 * _NINP
_B = _BATCH * _SEQ
_BPW = _B // _NW
_CH = 64
_PCH = _CH // 2
_NCHUNKS = _BPW // _CH
_NVEC = _NINP // 16
_LAST = _NINP - 16


@functools.partial(
    pl.kernel,
    mesh=plsc.VectorSubcoreMesh(core_axis_name="c", subcore_axis_name="s"),
    compiler_params=pltpu.CompilerParams(use_tc_tiling_on_sc=False),
    out_type=jax.ShapeDtypeStruct((_B // 2, _PW), jnp.float32),
    scratch_types=[
        pltpu.VMEM((_NCHUNKS, _CH), jnp.int32),
        pltpu.VMEM((_CH, _DP), jnp.float32),
        pltpu.VMEM((_CH, _DP), jnp.float32),
        pltpu.VMEM((_PCH, _PW), jnp.float32),
        pltpu.VMEM((_PCH, _PW), jnp.float32),
        pltpu.SemaphoreType.DMA,
        pltpu.SemaphoreType.DMA,
        pltpu.SemaphoreType.DMA,
        pltpu.SemaphoreType.DMA,
    ],
)
def _gather_kernel(idx_hbm, table_hbm, out_hbm, idx_t, rows0, rows1,
                   pack0, pack1, sg0, sg1, sw0, sw1):
    wid = lax.axis_index("s") * _NC + lax.axis_index("c")
    pbase = wid * (_BPW // 2)
    rows = (rows0, rows1)
    pack = (pack0, pack1)
    sg = (sg0, sg1)
    sw = (sw0, sw1)

    def g_start(c, b):
        pltpu.async_copy(table_hbm.at[idx_t.at[c]], rows[b], sg[b])

    def g_wait(b):
        pltpu.make_async_copy(table_hbm.at[idx_t.at[0]], rows[b],
                              sg[b]).wait()

    def w_start(c, b):
        pltpu.async_copy(pack[b], out_hbm.at[pl.ds(pbase + c * _PCH, _PCH)],
                         sw[b])

    def w_wait(b):
        pltpu.make_async_copy(pack[b], out_hbm.at[pl.ds(pbase, _PCH)],
                              sw[b]).wait()

    def compact(b):
        def cbody(r, carry):
            pr = lax.shift_right_logical(r, 1)
            off = lax.mul(lax.bitwise_and(r, 1), _NINP)
            for k in range(_NVEC):
                v = rows[b][r, pl.ds(16 * k, 16)]
                pack[b][pr, pl.ds(off + 16 * k, 16)] = v
            v = rows[b][r, pl.ds(_LAST, 16)]
            pack[b][pr, pl.ds(off + _LAST, 16)] = v
            return carry

        lax.fori_loop(0, _CH, cbody, 0)

    pltpu.sync_copy(idx_hbm.at[wid], idx_t)
    g_start(0, 0)
    g_start(1, 1)

    def step(c, b, drain, refill):
        g_wait(b)
        if drain:
            w_wait(b)
        compact(b)
        w_start(c, b)
        if refill:
            g_start(c + 2, b)

    step(0, 0, False, True)
    step(1, 1, False, True)

    def body(i, carry):
        step(2 * i, 0, True, True)
        step(2 * i + 1, 1, True, True)
        return carry

    lax.fori_loop(1, _NCHUNKS // 2 - 1, body, 0)

    step(_NCHUNKS - 2, 0, True, False)
    step(_NCHUNKS - 1, 1, True, False)
    w_wait(0)
    w_wait(1)


def kernel(input, table):
    idx = input.astype(jnp.int32).reshape(_NW, _NCHUNKS, _CH)
    table_p = jnp.pad(table, ((0, 0), (0, _DP - _NINP)))
    out = _gather_kernel(idx, table_p)
    return out.reshape(_BATCH, _SEQ, _NINP)


# speedup vs baseline: 1.1157x; 1.1157x over previous
import functools

import jax
import jax.numpy as jnp
from jax import lax
from jax.experimental import pallas as pl
from jax.experimental.pallas import tpu as pltpu
from jax.experimental.pallas import tpu_sc as plsc

_NTOKEN = 100000
_NINP = 300
_BATCH = 16384
_SEQ = 50

_NC = 2
_NS = 16
_NW = _NC * _NS

_DP = 304
_B = _BATCH * _SEQ
_BPW = _B // _NW
_CH = 64
_NCHUNKS = _BPW // _CH
_NSLOT = 4


@functools.partial(
    pl.kernel,
    mesh=plsc.VectorSubcoreMesh(core_axis_name="c", subcore_axis_name="s"),
    compiler_params=pltpu.CompilerParams(use_tc_tiling_on_sc=False),
    out_type=jax.ShapeDtypeStruct((_B, _DP), jnp.float32),
    scratch_types=[
        pltpu.VMEM((_BPW,), jnp.int32),
        pltpu.VMEM((_CH, _DP), jnp.float32),
        pltpu.VMEM((_CH, _DP), jnp.float32),
        pltpu.VMEM((_CH, _DP), jnp.float32),
        pltpu.VMEM((_CH, _DP), jnp.float32),
        pltpu.SemaphoreType.DMA,
        pltpu.SemaphoreType.DMA,
        pltpu.SemaphoreType.DMA,
        pltpu.SemaphoreType.DMA,
        pltpu.SemaphoreType.DMA,
        pltpu.SemaphoreType.DMA,
        pltpu.SemaphoreType.DMA,
        pltpu.SemaphoreType.DMA,
    ],
)
def _gather_kernel(idx_hbm, table_hbm, out_hbm, idx_t, r0, r1, r2, r3,
                   sg0, sg1, sg2, sg3, sw0, sw1, sw2, sw3):
    wid = lax.axis_index("s") * _NC + lax.axis_index("c")
    base = wid * _BPW
    rows = (r0, r1, r2, r3)
    sg = (sg0, sg1, sg2, sg3)
    sw = (sw0, sw1, sw2, sw3)

    def g_start(c, s):
        pltpu.async_copy(table_hbm.at[idx_t.at[pl.ds(c * _CH, _CH)]],
                         rows[s], sg[s])

    def g_wait(s):
        pltpu.make_async_copy(table_hbm.at[idx_t.at[pl.ds(0, _CH)]],
                              rows[s], sg[s]).wait()

    def w_start(c, s):
        pltpu.async_copy(rows[s], out_hbm.at[pl.ds(base + c * _CH, _CH)],
                         sw[s])

    def w_wait(s):
        pltpu.make_async_copy(rows[s], out_hbm.at[pl.ds(base, _CH)],
                              sw[s]).wait()

    pltpu.sync_copy(idx_hbm.at[pl.ds(base, _BPW)], idx_t)
    g_start(0, 0)
    g_start(1, 1)

    def step(c, s, drain, refill):
        g_wait(s)
        w_start(c, s)
        if drain:
            w_wait((s + 2) % _NSLOT)
        if refill:
            g_start(c + 2, (s + 2) % _NSLOT)

    step(0, 0, False, True)
    step(1, 1, False, True)

    def body(j, carry):
        c = 4 * j + 2
        for k in range(4):
            step(c + k, (2 + k) % _NSLOT, True, True)
        return carry

    lax.fori_loop(0, (_NCHUNKS - 4) // 4, body, 0)

    step(_NCHUNKS - 2, (_NCHUNKS - 2) % _NSLOT, True, False)
    step(_NCHUNKS - 1, (_NCHUNKS - 1) % _NSLOT, True, False)
    w_wait((_NCHUNKS - 2) % _NSLOT)
    w_wait((_NCHUNKS - 1) % _NSLOT)


def kernel(input, table):
    idx = input.astype(jnp.int32).reshape(_B)
    table_p = jnp.pad(table, ((0, 0), (0, _DP - _NINP)))
    out = _gather_kernel(idx, table_p)
    return out[:, :_NINP].reshape(_BATCH, _SEQ, _NINP)
